# Initial kernel scaffold; baseline (speedup 1.0000x reference)
import jax
import jax.numpy as jnp
from jax.experimental import pallas as pl


def kernel(x, edge_index, W, b):
    raise NotImplementedError("write your pallas kernel here")


# speedup vs baseline: 15.4014x; 15.4014x over previous
import functools

import jax
import jax.numpy as jnp
from jax import lax
from jax.experimental import pallas as pl
from jax.experimental.pallas import tpu as pltpu
from jax.experimental.pallas import tpu_sc as plsc

D = 128
NC, NS = 2, 16
NW = NC * NS
EB = 128

_mesh = functools.partial(
    plsc.VectorSubcoreMesh, core_axis_name="c", subcore_axis_name="s",
    num_cores=NC, num_subcores=NS)


def _wid():
    return lax.axis_index("c") * NS + lax.axis_index("s")


def _fill(ref, n, value, dtype):
    v = jnp.full((16,), value, dtype)
    for i in range(n // 16):
        ref[pl.ds(i * 16, 16)] = v


def _deg_call(dst_p, npad, ept):
    nb = ept // EB
    rpt = npad // NS

    def body(dst_hbm, degp_hbm, didx_v, ones_v, zero_v, deg_sh):
        c = lax.axis_index("c")
        s = lax.axis_index("s")
        wid = c * NS + s
        _fill(ones_v, EB, 1.0, jnp.float32)
        _fill(zero_v, rpt, 0.0, jnp.float32)
        r0 = s * rpt
        pltpu.sync_copy(zero_v, deg_sh.at[pl.ds(r0, rpt)])
        plsc.subcore_barrier()

        def step(i, _):
            off = wid * ept + i * EB
            pltpu.sync_copy(dst_hbm.at[pl.ds(off, EB)], didx_v)
            pltpu.sync_copy(ones_v, deg_sh.at[didx_v], add=True)
            return 0

        lax.fori_loop(0, nb, step, 0)
        plsc.subcore_barrier()
        pltpu.sync_copy(deg_sh.at[pl.ds(r0, rpt)],
                        degp_hbm.at[c, pl.ds(r0, rpt)])

    return pl.kernel(
        body,
        out_type=jax.ShapeDtypeStruct((NC, npad), jnp.float32),
        mesh=_mesh(),
        scratch_types=[
            pltpu.VMEM((EB,), jnp.int32),
            pltpu.VMEM((EB,), jnp.float32),
            pltpu.VMEM((rpt,), jnp.float32),
            pltpu.VMEM_SHARED((npad,), jnp.float32),
        ],
    )(dst_p)


def _agg_call(src_p, dst_p, y, npad, ept):
    nb = ept // EB
    rpt = npad // NS

    def body(src_hbm, dst_hbm, y_hbm, out_hbm, sidx_v, didx_v, rows_v, acc_sh,
             sem):
        c = lax.axis_index("c")
        s = lax.axis_index("s")
        wid = c * NS + s
        r0 = s * rpt
        pltpu.sync_copy(y_hbm.at[pl.ds(r0, rpt)], acc_sh.at[pl.ds(r0, rpt)])
        plsc.subcore_barrier()

        def step(i, _):
            off = wid * ept + i * EB
            pltpu.sync_copy(src_hbm.at[pl.ds(off, EB)], sidx_v)
            pltpu.sync_copy(dst_hbm.at[pl.ds(off, EB)], didx_v)
            pltpu.async_copy(y_hbm.at[sidx_v], rows_v, sem).wait()
            pltpu.sync_copy(rows_v, acc_sh.at[didx_v], add=True)
            return 0

        lax.fori_loop(0, nb, step, 0)
        plsc.subcore_barrier()
        pltpu.sync_copy(acc_sh.at[pl.ds(r0, rpt)],
                        out_hbm.at[c, pl.ds(r0, rpt)])

    return pl.kernel(
        body,
        out_type=jax.ShapeDtypeStruct((NC, npad, D), jnp.float32),
        mesh=_mesh(),
        scratch_types=[
            pltpu.VMEM((EB,), jnp.int32),
            pltpu.VMEM((EB,), jnp.int32),
            pltpu.VMEM((EB, D), jnp.float32),
            pltpu.VMEM_SHARED((npad, D), jnp.float32),
            pltpu.SemaphoreType.DMA,
        ],
    )(src_p, dst_p, y)


def _y_call(xp, W, degp, npad):
    rb = 1024
    grid = npad // rb

    def body(x_ref, w_ref, degp_ref, y_ref):
        deg = degp_ref[0, :] + degp_ref[1, :] + 1.0
        dinv = lax.rsqrt(deg)
        xw = jnp.dot(x_ref[...], w_ref[...],
                     preferred_element_type=jnp.float32)
        y_ref[...] = xw * dinv[:, None]

    return pl.pallas_call(
        body,
        out_shape=jax.ShapeDtypeStruct((npad, D), jnp.float32),
        grid=(grid,),
        in_specs=[
            pl.BlockSpec((rb, D), lambda j: (j, 0)),
            pl.BlockSpec((D, D), lambda j: (0, 0)),
            pl.BlockSpec((NC, rb), lambda j: (0, j)),
        ],
        out_specs=pl.BlockSpec((rb, D), lambda j: (j, 0)),
    )(xp, W, degp)


def _final_call(P, y, degp, b2, npad):
    rb = 1024
    grid = npad // rb

    def body(p_ref, y_ref, degp_ref, b_ref, o_ref):
        deg = degp_ref[0, :] + degp_ref[1, :] + 1.0
        dinv = lax.rsqrt(deg)
        ssum = p_ref[0, :, :] + p_ref[1, :, :] - y_ref[...]
        o_ref[...] = jnp.maximum(ssum * dinv[:, None] + b_ref[0, :][None, :],
                                 0.0)

    return pl.pallas_call(
        body,
        out_shape=jax.ShapeDtypeStruct((npad, D), jnp.float32),
        grid=(grid,),
        in_specs=[
            pl.BlockSpec((NC, rb, D), lambda j: (0, j, 0)),
            pl.BlockSpec((rb, D), lambda j: (j, 0)),
            pl.BlockSpec((NC, rb), lambda j: (0, j)),
            pl.BlockSpec((1, D), lambda j: (0, 0)),
        ],
        out_specs=pl.BlockSpec((rb, D), lambda j: (j, 0)),
    )(P, y, degp, b2)


def kernel(x, edge_index, W, b):
    n = x.shape[0]
    e = edge_index.shape[1]
    npad = ((n + 1024) // 1024) * 1024
    ept = ((e + NW - 1) // NW + EB - 1) // EB * EB
    epad = ept * NW

    src = edge_index[0].astype(jnp.int32)
    dst = edge_index[1].astype(jnp.int32)
    src_p = jnp.concatenate([src, jnp.zeros((epad - e,), jnp.int32)])
    dst_p = jnp.concatenate([dst, jnp.full((epad - e,), n, jnp.int32)])
    xp = jnp.pad(x, ((0, npad - n), (0, 0)))

    degp = _deg_call(dst_p, npad, ept)
    y = _y_call(xp, W, degp, npad)
    P = _agg_call(src_p, dst_p, y, npad, ept)
    return _final_call(P, y, degp, b.reshape(1, D), npad)[:n]
